# baseline (speedup 1.0000x reference)
import jax
import jax.numpy as jnp
from jax.experimental import pallas as pl
from jax.experimental.pallas import tpu as pltpu

_B, _C, _T = 16, 128, 1500
_BINS = 2048
_CDIM = 32
_NQ = 8
_NGRP = _BINS // 128

_MM = jax.lax.Precision.DEFAULT


def _mm(a, b):
    return jax.lax.dot_general(a, b, (((1,), (0,)), ((), ())),
                               precision=_MM,
                               preferred_element_type=jnp.float32)


def _gather_rows(eT, idx, rows):
    r = jnp.broadcast_to(idx % 128, (rows, idx.shape[1]))
    g = idx // 128
    acc = None
    for gi in range(_NGRP):
        part = jnp.take_along_axis(eT[:, gi * 128:(gi + 1) * 128], r, axis=1,
                                   mode="promise_in_bounds")
        mask = (g == gi).astype(jnp.float32)
        acc = part * mask if acc is None else acc + part * mask
    return acc


def _vq_body(x_ref, w_in_f_ref, w_in_r_ref, b_in_f_ref, b_in_r_ref,
             cb_f_ref, cb_r_ref, w_out_f_ref, w_out_r_ref,
             b_out_f_ref, b_out_r_ref,
             emb_ref, codes_ref, loss_ref, cbm2_ref, cbT_ref):
    b = pl.program_id(0)

    def cb_q(q):
        return cb_f_ref[0] if q == 0 else cb_r_ref[q - 1]

    @pl.when(b == 0)
    def _init():
        loss_ref[...] = jnp.zeros((1, 1), jnp.float32)
        for q in range(_NQ):
            e = cb_q(q)
            cbm2_ref[q] = -2.0 * e
            cbT_ref[q] = e.T

    xt = x_ref[0]
    residual = xt
    emb = jnp.zeros_like(xt)
    idx_rows = []
    loss = jnp.float32(0.0)
    for q in range(_NQ):
        if q == 1:
            residual = xt
        w_in = w_in_f_ref[0] if q == 0 else w_in_r_ref[q - 1]
        b_in = b_in_f_ref[0] if q == 0 else b_in_r_ref[q - 1]
        w_out = w_out_f_ref[0] if q == 0 else w_out_r_ref[q - 1]
        b_out = b_out_f_ref[0] if q == 0 else b_out_r_ref[q - 1]
        v = _mm(w_in, residual) + b_in
        e = cb_q(q)
        ee = jnp.sum(e * e, axis=1, keepdims=True)
        vme2 = _mm(cbm2_ref[q], v)
        vv = jnp.sum(v * v, axis=0, keepdims=True)
        d = (vv + vme2) + ee
        idx = jnp.argmin(d, axis=0, keepdims=True)
        quant = _gather_rows(cbT_ref[q], idx, _CDIM)
        diff = quant - v
        loss = loss + jnp.sum(diff * diff)
        out = _mm(w_out, quant) + b_out
        residual = residual - out
        emb = emb + out
        idx_rows.append(idx.astype(jnp.int32))
    emb_ref[0] = emb
    codes_ref[0] = jnp.concatenate(idx_rows, axis=0)
    loss_ref[...] = loss_ref[...] + loss


def kernel(x, W_in_first, b_in_first, codebook_first, W_out_first, b_out_first,
           W_in_rest, b_in_rest, codebook_rest, W_out_rest, b_out_rest):
    full = lambda *shape: pl.BlockSpec(shape, lambda b: (0,) * len(shape))
    emb, codes, loss = pl.pallas_call(
        _vq_body,
        grid=(_B,),
        in_specs=[
            pl.BlockSpec((1, _C, _T), lambda b: (b, 0, 0)),
            full(1, _CDIM, _C),
            full(_NQ - 1, _CDIM, _C),
            full(1, _CDIM, 1),
            full(_NQ - 1, _CDIM, 1),
            full(1, _BINS, _CDIM),
            full(_NQ - 1, _BINS, _CDIM),
            full(1, _C, _CDIM),
            full(_NQ - 1, _C, _CDIM),
            full(1, _C, 1),
            full(_NQ - 1, _C, 1),
        ],
        out_specs=[
            pl.BlockSpec((1, _C, _T), lambda b: (b, 0, 0)),
            pl.BlockSpec((1, _NQ, _T), lambda b: (b, 0, 0)),
            pl.BlockSpec((1, 1), lambda b: (0, 0)),
        ],
        out_shape=[
            jax.ShapeDtypeStruct((_B, _C, _T), jnp.float32),
            jax.ShapeDtypeStruct((_B, _NQ, _T), jnp.int32),
            jax.ShapeDtypeStruct((1, 1), jnp.float32),
        ],
        scratch_shapes=[pltpu.VMEM((_NQ, _BINS, _CDIM), jnp.float32),
                        pltpu.VMEM((_NQ, _CDIM, _BINS), jnp.float32)],
    )(x, W_in_first, W_in_rest, b_in_first[:, :, None], b_in_rest[:, :, None],
      codebook_first, codebook_rest, W_out_first, W_out_rest,
      b_out_first[:, :, None], b_out_rest[:, :, None])

    full_quantized_emb = jnp.transpose(emb, (0, 2, 1))
    full_quantized_codes = jnp.transpose(codes, (0, 2, 1))
    penalty = loss[0, 0] / jnp.float32(_B * _T * _CDIM * _NQ)
    return full_quantized_emb, full_quantized_codes, penalty


# speedup vs baseline: 1.0267x; 1.0267x over previous
import jax
import jax.numpy as jnp
from jax.experimental import pallas as pl
from jax.experimental.pallas import tpu as pltpu

_B, _C, _T = 16, 128, 1500
_BINS = 2048
_CDIM = 32
_NQ = 8
_NGRP = _BINS // 128

_MM = jax.lax.Precision.DEFAULT


def _mm(a, b):
    return jax.lax.dot_general(a, b, (((1,), (0,)), ((), ())),
                               precision=_MM,
                               preferred_element_type=jnp.float32)


def _gather_rows(eT, idx, rows):
    r = jnp.broadcast_to(idx % 128, (rows, idx.shape[1]))
    g = idx // 128
    acc = None
    for gi in range(_NGRP):
        part = jnp.take_along_axis(eT[:, gi * 128:(gi + 1) * 128], r, axis=1,
                                   mode="promise_in_bounds")
        mask = (g == gi).astype(jnp.float32)
        acc = part * mask if acc is None else acc + part * mask
    return acc


def _vq_body(x_ref, w_in_f_ref, w_in_r_ref, b_in_f_ref, b_in_r_ref,
             cb_f_ref, cb_r_ref, w_out_f_ref, w_out_r_ref,
             b_out_f_ref, b_out_r_ref,
             emb_ref, codes_ref, loss_ref, cbm2_ref, cbT_ref):
    b = pl.program_id(0)

    def cb_q(q):
        return cb_f_ref[0] if q == 0 else cb_r_ref[q - 1]

    @pl.when(b == 0)
    def _init():
        loss_ref[...] = jnp.zeros((1, 1), jnp.float32)
        for q in range(_NQ):
            e = cb_q(q)
            cbm2_ref[q] = -2.0 * e
            cbT_ref[q] = e.T

    xt = x_ref[0]
    residual = xt
    emb = jnp.zeros_like(xt)
    idx_rows = []
    loss = jnp.float32(0.0)
    for q in range(_NQ):
        if q == 1:
            residual = xt
        w_in = w_in_f_ref[0] if q == 0 else w_in_r_ref[q - 1]
        b_in = b_in_f_ref[0] if q == 0 else b_in_r_ref[q - 1]
        w_out = w_out_f_ref[0] if q == 0 else w_out_r_ref[q - 1]
        b_out = b_out_f_ref[0] if q == 0 else b_out_r_ref[q - 1]
        v = _mm(w_in, residual) + b_in
        e = cb_q(q)
        ee = jnp.sum(e * e, axis=1, keepdims=True)
        vme2 = _mm(cbm2_ref[q], v)
        vv = jnp.sum(v * v, axis=0, keepdims=True)
        d = (vv + vme2) + ee
        idx = jnp.argmin(d, axis=0, keepdims=True)
        quant = _gather_rows(cbT_ref[q], idx, _CDIM)
        diff = quant - v
        loss = loss + jnp.sum(diff * diff)
        out = _mm(w_out, quant) + b_out
        residual = residual - out
        emb = emb + out
        idx_rows.append(idx.astype(jnp.int32))
    emb_ref[0] = emb.T
    codes_ref[0] = jnp.concatenate(idx_rows, axis=0)
    loss_ref[...] = loss_ref[...] + loss


def kernel(x, W_in_first, b_in_first, codebook_first, W_out_first, b_out_first,
           W_in_rest, b_in_rest, codebook_rest, W_out_rest, b_out_rest):
    full = lambda *shape: pl.BlockSpec(shape, lambda b: (0,) * len(shape))
    emb, codes, loss = pl.pallas_call(
        _vq_body,
        grid=(_B,),
        in_specs=[
            pl.BlockSpec((1, _C, _T), lambda b: (b, 0, 0)),
            full(1, _CDIM, _C),
            full(_NQ - 1, _CDIM, _C),
            full(1, _CDIM, 1),
            full(_NQ - 1, _CDIM, 1),
            full(1, _BINS, _CDIM),
            full(_NQ - 1, _BINS, _CDIM),
            full(1, _C, _CDIM),
            full(_NQ - 1, _C, _CDIM),
            full(1, _C, 1),
            full(_NQ - 1, _C, 1),
        ],
        out_specs=[
            pl.BlockSpec((1, _T, _C), lambda b: (b, 0, 0)),
            pl.BlockSpec((1, _NQ, _T), lambda b: (b, 0, 0)),
            pl.BlockSpec((1, 1), lambda b: (0, 0)),
        ],
        out_shape=[
            jax.ShapeDtypeStruct((_B, _T, _C), jnp.float32),
            jax.ShapeDtypeStruct((_B, _NQ, _T), jnp.int32),
            jax.ShapeDtypeStruct((1, 1), jnp.float32),
        ],
        scratch_shapes=[pltpu.VMEM((_NQ, _BINS, _CDIM), jnp.float32),
                        pltpu.VMEM((_NQ, _CDIM, _BINS), jnp.float32)],
    )(x, W_in_first, W_in_rest, b_in_first[:, :, None], b_in_rest[:, :, None],
      codebook_first, codebook_rest, W_out_first, W_out_rest,
      b_out_first[:, :, None], b_out_rest[:, :, None])

    full_quantized_emb = emb
    full_quantized_codes = jnp.transpose(codes, (0, 2, 1))
    penalty = loss[0, 0] / jnp.float32(_B * _T * _CDIM * _NQ)
    return full_quantized_emb, full_quantized_codes, penalty
